# baseline (speedup 1.0000x reference)
import functools

import jax
import jax.numpy as jnp
from jax import lax
from jax.experimental import pallas as pl
from jax.experimental.pallas import tpu as pltpu
from jax.experimental.pallas import tpu_sc as plsc

B, S, D = 1, 2048, 2048
H, Dh = 16, 128
E, TOPK = 8, 2
MOE_I, SHARED_I = 1408, 5632
EPS, THETA = 1e-6, 10000.0

BT = 256
BQ = 2048
BK = 512
RQK = BQ // BK
BN = 256
BM = 128
NT = S * TOPK // BM + E
NTOT = NT * BM

_VMEM_BIG = pltpu.CompilerParams(vmem_limit_bytes=66846720)


def _rmsnorm_body(x_ref, w_ref, o_ref):
    x = x_ref[...]
    v = jnp.mean(x * x, axis=1, keepdims=True)
    o_ref[...] = x * lax.rsqrt(v + EPS) * w_ref[...]


def _rmsnorm(x, w):
    return pl.pallas_call(
        _rmsnorm_body,
        grid=(S // BT,),
        in_specs=[pl.BlockSpec((BT, D), lambda i: (i, 0)),
                  pl.BlockSpec((1, D), lambda i: (0, 0))],
        out_specs=pl.BlockSpec((BT, D), lambda i: (i, 0)),
        out_shape=jax.ShapeDtypeStruct((S, D), jnp.float32),
    )(x, w.reshape(1, D))


def _rope_tables_body(cos_ref, sin_ref):
    pos = lax.broadcasted_iota(jnp.int32, (S, 1), 0).astype(jnp.float32)
    inv = THETA ** (
        -2.0 * lax.broadcasted_iota(jnp.int32, (1, Dh // 2), 1).astype(jnp.float32) / Dh)
    freqs = pos * inv
    cos_h, sin_h = jnp.cos(freqs), jnp.sin(freqs)
    cos_ref[...] = jnp.concatenate([cos_h, cos_h], axis=1)
    sin_ref[...] = jnp.concatenate([sin_h, sin_h], axis=1)


def _rope_tables():
    return pl.pallas_call(
        _rope_tables_body,
        out_shape=[jax.ShapeDtypeStruct((S, Dh), jnp.float32),
                   jax.ShapeDtypeStruct((S, Dh), jnp.float32)],
    )()


BNQ = 256


def _apply_rope(acc, cos, sin):
    outs = []
    for h in range(BNQ // Dh):
        qh = acc[:, h * Dh:(h + 1) * Dh]
        x1, x2 = qh[:, :Dh // 2], qh[:, Dh // 2:]
        rot = jnp.concatenate([-x2, x1], axis=1)
        outs.append(qh * cos + rot * sin)
    return jnp.concatenate(outs, axis=1)


def _qkv_body(x_ref, ln_ref, qw_ref, kw_ref, vw_ref, cos_ref, sin_ref,
              q_ref, k_ref, v_ref):
    x = x_ref[...]
    var = jnp.mean(x * x, axis=1, keepdims=True)
    xn = x * lax.rsqrt(var + EPS) * ln_ref[...]
    cos, sin = cos_ref[...], sin_ref[...]
    q = jnp.dot(xn, qw_ref[...], preferred_element_type=jnp.float32)
    q_ref[...] = _apply_rope(q, cos, sin)
    k = jnp.dot(xn, kw_ref[...], preferred_element_type=jnp.float32)
    k_ref[...] = _apply_rope(k, cos, sin)
    v_ref[...] = jnp.dot(xn, vw_ref[...], preferred_element_type=jnp.float32)


def _qkv_project(x, ln_w, q_w, k_w, v_w, cos, sin):
    wspec = pl.BlockSpec((D, BNQ), lambda j: (0, j))
    ospec = pl.BlockSpec((S, BNQ), lambda j: (0, j))
    tspec = pl.BlockSpec((S, Dh), lambda j: (0, 0))
    return pl.pallas_call(
        _qkv_body,
        grid=(D // BNQ,),
        in_specs=[pl.BlockSpec((S, D), lambda j: (0, 0)),
                  pl.BlockSpec((1, D), lambda j: (0, 0)),
                  wspec, wspec, wspec, tspec, tspec],
        out_specs=[ospec, ospec, ospec],
        out_shape=[jax.ShapeDtypeStruct((S, D), jnp.float32)] * 3,
        compiler_params=_VMEM_BIG,
    )(x, ln_w.reshape(1, D), q_w, k_w, v_w, cos, sin)


def _flash_body(qk_ref, q_ref, k_ref, v_ref, o_ref, acc_ref, m_ref, l_ref):
    t = pl.program_id(1)
    qi, ki = qk_ref[0, t], qk_ref[1, t]

    @pl.when(ki == 0)
    def _():
        acc_ref[...] = jnp.zeros_like(acc_ref)
        m_ref[...] = jnp.full_like(m_ref, -jnp.inf)
        l_ref[...] = jnp.zeros_like(l_ref)

    q, k, v = q_ref[...], k_ref[...], v_ref[...]
    s = lax.dot_general(q, k, (((1,), (1,)), ((), ())),
                        preferred_element_type=jnp.float32)
    s = s * (1.0 / (Dh ** 0.5))

    def _update(s):
        m_prev, l_prev = m_ref[...], l_ref[...]
        m_cur = jnp.max(s, axis=1, keepdims=True)
        m_new = jnp.maximum(m_prev, m_cur)
        alpha = jnp.exp(m_prev - m_new)
        p = jnp.exp(s - m_new[:, 0:1])
        l_ref[...] = alpha * l_prev + jnp.sum(p, axis=1, keepdims=True)
        acc_ref[...] = alpha * acc_ref[...] + jnp.dot(
            p, v, preferred_element_type=jnp.float32)
        m_ref[...] = m_new

    @pl.when(ki < qi * RQK)
    def _():
        _update(s)

    @pl.when(ki >= qi * RQK)
    def _():
        row = qi * BQ + lax.broadcasted_iota(jnp.int32, (BQ, BK), 0)
        col = ki * BK + lax.broadcasted_iota(jnp.int32, (BQ, BK), 1)
        _update(jnp.where(col <= row, s, -1e30))

    @pl.when(ki == (qi + 1) * RQK - 1)
    def _():
        o_ref[...] = acc_ref[...] / l_ref[...]


def _flash_attention(q, k, v):
    nq = S // BQ
    tri = [(qi, ki) for qi in range(nq) for ki in range((qi + 1) * RQK)]
    qk = jnp.array([[a for a, _ in tri], [b for _, b in tri]], jnp.int32)
    spec = pltpu.PrefetchScalarGridSpec(
        num_scalar_prefetch=1,
        grid=(H, len(tri)),
        in_specs=[
            pl.BlockSpec((BQ, Dh), lambda h, t, qk: (qk[0, t], h)),
            pl.BlockSpec((BK, Dh), lambda h, t, qk: (qk[1, t], h)),
            pl.BlockSpec((BK, Dh), lambda h, t, qk: (qk[1, t], h)),
        ],
        out_specs=pl.BlockSpec((BQ, Dh), lambda h, t, qk: (qk[0, t], h)),
        scratch_shapes=[pltpu.VMEM((BQ, Dh), jnp.float32),
                        pltpu.VMEM((BQ, 128), jnp.float32),
                        pltpu.VMEM((BQ, 128), jnp.float32)],
    )
    return pl.pallas_call(
        _flash_body,
        grid_spec=spec,
        out_shape=jax.ShapeDtypeStruct((S, H * Dh), jnp.float32),
    )(qk, q, k, v)


def _mm_res_body(x_ref, w_ref, r_ref, o_ref):
    o_ref[...] = r_ref[...] + jnp.dot(x_ref[...], w_ref[...],
                                      preferred_element_type=jnp.float32)


def _o_proj_residual(ctx, o_w, residual):
    return pl.pallas_call(
        _mm_res_body,
        grid=(D // BN,),
        in_specs=[pl.BlockSpec((S, H * Dh), lambda j: (0, 0)),
                  pl.BlockSpec((H * Dh, BN), lambda j: (0, j)),
                  pl.BlockSpec((S, BN), lambda j: (0, j))],
        out_specs=pl.BlockSpec((S, BN), lambda j: (0, j)),
        out_shape=jax.ShapeDtypeStruct((S, D), jnp.float32),
        compiler_params=_VMEM_BIG,
    )(ctx, o_w, residual)


def _router_body(x_ref, ln_ref, rw_ref, gw_ref,
                 x2_ref, tv_ref, ti_ref, sig_ref):
    x = x_ref[...]
    v = jnp.mean(x * x, axis=1, keepdims=True)
    xn = x * lax.rsqrt(v + EPS) * ln_ref[...]
    x2_ref[...] = xn
    logits = jnp.dot(xn, rw_ref[...], preferred_element_type=jnp.float32)
    m = jnp.max(logits, axis=1, keepdims=True)
    e = jnp.exp(logits - m)
    p = e / jnp.sum(e, axis=1, keepdims=True)
    ii = lax.broadcasted_iota(jnp.int32, p.shape, 1)
    m1 = jnp.max(p, axis=1, keepdims=True)
    i1 = jnp.min(jnp.where(p == m1, ii, E), axis=1, keepdims=True)
    p2 = jnp.where(ii == i1, -1.0, p)
    m2 = jnp.max(p2, axis=1, keepdims=True)
    i2 = jnp.min(jnp.where(p2 == m2, ii, E), axis=1, keepdims=True)
    tv_ref[...] = jnp.concatenate([m1, m2], axis=1)
    ti_ref[...] = jnp.concatenate([i1, i2], axis=1)
    sg = jnp.dot(xn, gw_ref[...], preferred_element_type=jnp.float32)
    sig_ref[...] = jax.nn.sigmoid(sg)


def _router(hidden, ln2_w, router_w, sgate_w):
    return pl.pallas_call(
        _router_body,
        grid=(S // BT,),
        in_specs=[pl.BlockSpec((BT, D), lambda i: (i, 0)),
                  pl.BlockSpec((1, D), lambda i: (0, 0)),
                  pl.BlockSpec((D, E), lambda i: (0, 0)),
                  pl.BlockSpec((D, 1), lambda i: (0, 0))],
        out_specs=[pl.BlockSpec((BT, D), lambda i: (i, 0)),
                   pl.BlockSpec((BT, TOPK), lambda i: (i, 0)),
                   pl.BlockSpec((BT, TOPK), lambda i: (i, 0)),
                   pl.BlockSpec((BT, 1), lambda i: (i, 0))],
        out_shape=[jax.ShapeDtypeStruct((S, D), jnp.float32),
                   jax.ShapeDtypeStruct((S, TOPK), jnp.float32),
                   jax.ShapeDtypeStruct((S, TOPK), jnp.int32),
                   jax.ShapeDtypeStruct((S, 1), jnp.float32)],
    )(hidden, ln2_w.reshape(1, D), router_w, sgate_w)


def _sc_gather_rows(table, idx, total):
    info = plsc.get_sparse_core_info()
    nw = info.num_cores * info.num_subcores
    bpw = total // nw
    ch = 16
    dm = table.shape[1]
    mesh = plsc.VectorSubcoreMesh(core_axis_name="c", subcore_axis_name="s")

    @functools.partial(
        pl.kernel, mesh=mesh,
        out_type=jax.ShapeDtypeStruct((total, dm), jnp.float32),
        scratch_types=[pltpu.VMEM((bpw,), jnp.int32),
                       pltpu.VMEM((ch, dm), jnp.float32),
                       pltpu.VMEM((ch, dm), jnp.float32),
                       pltpu.SemaphoreType.DMA,
                       pltpu.SemaphoreType.DMA],
    )
    def gather(table_hbm, idx_hbm, out_hbm, idx_v, rows_a, rows_b, sem_a, sem_b):
        wid = lax.axis_index("s") * info.num_cores + lax.axis_index("c")
        base = wid * bpw
        pltpu.sync_copy(idx_hbm.at[pl.ds(base, bpw)], idx_v)
        bufs = (rows_a, rows_b)
        sems = (sem_a, sem_b)
        nch = bpw // ch
        cps = [None, None]
        for c in range(nch):
            cps[c % 2] = pltpu.async_copy(
                table_hbm.at[idx_v.at[pl.ds(c * ch, ch)]], bufs[c % 2],
                sems[c % 2])
            if c > 0:
                cps[(c - 1) % 2].wait()
                pltpu.sync_copy(bufs[(c - 1) % 2],
                                out_hbm.at[pl.ds(base + (c - 1) * ch, ch)])
        cps[(nch - 1) % 2].wait()
        pltpu.sync_copy(bufs[(nch - 1) % 2],
                        out_hbm.at[pl.ds(base + (nch - 1) * ch, ch)])

    return gather(table, idx)


def _sc_scatter_rows(rows, idx3, total):
    info = plsc.get_sparse_core_info()
    nw = info.num_cores * info.num_subcores
    nwk, nch, ch = idx3.shape
    rpw = rows.shape[0] // nw
    dm = rows.shape[1]
    mesh = plsc.VectorSubcoreMesh(core_axis_name="c", subcore_axis_name="s")

    @functools.partial(
        pl.kernel, mesh=mesh,
        out_type=jax.ShapeDtypeStruct((total, dm), jnp.float32),
        scratch_types=[pltpu.VMEM((nch, ch), jnp.int32),
                       pltpu.VMEM((ch, dm), jnp.float32),
                       pltpu.VMEM((ch, dm), jnp.float32),
                       pltpu.SemaphoreType.DMA,
                       pltpu.SemaphoreType.DMA],
    )
    def scatter(rows_hbm, idx_hbm, out_hbm, idx_v, buf_a, buf_b, sem_a, sem_b):
        wid = lax.axis_index("s") * info.num_cores + lax.axis_index("c")
        base = wid * rpw
        pltpu.sync_copy(idx_hbm.at[wid], idx_v)
        bufs = (buf_a, buf_b)
        sems = (sem_a, sem_b)
        half = nch // 2
        cps = [None, None]
        for c in range(nch):
            if cps[c % 2] is not None:
                cps[c % 2].wait()
            src = base + (c % half) * ch
            pltpu.sync_copy(rows_hbm.at[pl.ds(src, ch)], bufs[c % 2])
            cps[c % 2] = pltpu.async_copy(
                bufs[c % 2], out_hbm.at[idx_v.at[c]], sems[c % 2])
        cps[(nch - 1) % 2].wait()
        if nch > 1:
            cps[(nch - 2) % 2].wait()

    return scatter(rows, idx3)


def _moe_gu_body(eid_ref, xg_ref, eg_ref, eu_ref, h_ref):
    x = xg_ref[...]
    g = jnp.dot(x, eg_ref[0], preferred_element_type=jnp.float32)
    u = jnp.dot(x, eu_ref[0], preferred_element_type=jnp.float32)
    h_ref[...] = g * jax.nn.sigmoid(g) * u


def _moe_down_body(eid_ref, h_ref, ed_ref, yg_ref):
    yg_ref[...] = jnp.dot(h_ref[...], ed_ref[0],
                          preferred_element_type=jnp.float32)


def _grouped_mlp(eid, xg, eg_w, eu_w, ed_w):
    gu_spec = pltpu.PrefetchScalarGridSpec(
        num_scalar_prefetch=1,
        grid=(NT,),
        in_specs=[
            pl.BlockSpec((BM, D), lambda t, eid: (t, 0)),
            pl.BlockSpec((1, D, MOE_I), lambda t, eid: (eid[t], 0, 0)),
            pl.BlockSpec((1, D, MOE_I), lambda t, eid: (eid[t], 0, 0)),
        ],
        out_specs=pl.BlockSpec((BM, MOE_I), lambda t, eid: (t, 0)),
    )
    h = pl.pallas_call(
        _moe_gu_body,
        grid_spec=gu_spec,
        out_shape=jax.ShapeDtypeStruct((NTOT, MOE_I), jnp.float32),
        compiler_params=_VMEM_BIG,
    )(eid, xg, eg_w, eu_w)
    down_spec = pltpu.PrefetchScalarGridSpec(
        num_scalar_prefetch=1,
        grid=(NT,),
        in_specs=[
            pl.BlockSpec((BM, MOE_I), lambda t, eid: (t, 0)),
            pl.BlockSpec((1, MOE_I, D), lambda t, eid: (eid[t], 0, 0)),
        ],
        out_specs=pl.BlockSpec((BM, D), lambda t, eid: (t, 0)),
    )
    return pl.pallas_call(
        _moe_down_body,
        grid_spec=down_spec,
        out_shape=jax.ShapeDtypeStruct((NTOT, D), jnp.float32),
        compiler_params=_VMEM_BIG,
    )(eid, h, ed_w)


def _shared_body(x_ref, g_ref, u_ref, d_ref, o_ref):
    j = pl.program_id(0)
    x = x_ref[...]
    g = jnp.dot(x, g_ref[...], preferred_element_type=jnp.float32)
    u = jnp.dot(x, u_ref[...], preferred_element_type=jnp.float32)
    h = g * jax.nn.sigmoid(g) * u
    y = jnp.dot(h, d_ref[...], preferred_element_type=jnp.float32)

    @pl.when(j == 0)
    def _():
        o_ref[...] = y

    @pl.when(j > 0)
    def _():
        o_ref[...] = o_ref[...] + y


BMS = 1024


def _shared_mlp_half(x2, sg_w, su_w, sd_w, half):
    return pl.pallas_call(
        _shared_body,
        grid=(SHARED_I // BN,),
        in_specs=[pl.BlockSpec((BMS, D), lambda j, h=half: (h, 0)),
                  pl.BlockSpec((D, BN), lambda j: (0, j)),
                  pl.BlockSpec((D, BN), lambda j: (0, j)),
                  pl.BlockSpec((BN, D), lambda j: (j, 0))],
        out_specs=pl.BlockSpec((BMS, D), lambda j: (0, 0)),
        out_shape=jax.ShapeDtypeStruct((BMS, D), jnp.float32),
        compiler_params=_VMEM_BIG,
    )(x2, sg_w, su_w, sd_w)


def _epilogue_body(r_ref, sha_ref, shb_ref, sig_ref, tv_ref, y1_ref, y2_ref,
                   o_ref):
    i = pl.program_id(0)
    w = tv_ref[...]
    sh = jnp.where(i < (BMS // BT), sha_ref[...], shb_ref[...])
    o_ref[...] = (r_ref[...] + sh * sig_ref[...]
                  + w[:, 0:1] * y1_ref[...] + w[:, 1:2] * y2_ref[...])


def _epilogue(residual, shared_a, shared_b, sig, tv, ypair):
    nb = S // BT
    hb = BMS // BT
    return pl.pallas_call(
        _epilogue_body,
        grid=(nb,),
        in_specs=[pl.BlockSpec((BT, D), lambda i: (i, 0)),
                  pl.BlockSpec((BT, D), lambda i, hb=hb: (jnp.minimum(i, hb - 1), 0)),
                  pl.BlockSpec((BT, D), lambda i, hb=hb: (jnp.maximum(i - hb, 0), 0)),
                  pl.BlockSpec((BT, 1), lambda i: (i, 0)),
                  pl.BlockSpec((BT, TOPK), lambda i: (i, 0)),
                  pl.BlockSpec((BT, D), lambda i: (i, 0)),
                  pl.BlockSpec((BT, D), lambda i, nb=nb: (i + nb, 0))],
        out_specs=pl.BlockSpec((BT, D), lambda i: (i, 0)),
        out_shape=jax.ShapeDtypeStruct((S, D), jnp.float32),
    )(residual, shared_a, shared_b, sig, tv, ypair, ypair)


def kernel(hidden_states, attention_mask, position_ids, causal_mask, params):
    x = hidden_states.reshape(S, D).astype(jnp.float32)

    cos, sin = _rope_tables()
    q, k, v = _qkv_project(x, params['ln1_w'], params['q_w'], params['k_w'],
                           params['v_w'], cos, sin)
    ctx = _flash_attention(q, k, v)
    hidden = _o_proj_residual(ctx, params['o_w'], x)

    x2, tv, ti, sig = _router(hidden, params['ln2_w'],
                              params['router_w'], params['sgate_w'])

    onehot = (ti[:, :, None] == jnp.arange(E)[None, None, :]).astype(jnp.int32)
    per_tok = onehot.sum(1)
    counts = per_tok.sum(0)
    excl = jnp.cumsum(per_tok, axis=0) - per_tok
    rank = jnp.take_along_axis(excl, ti, axis=1)
    tiles = (counts + BM - 1) // BM
    cum_tiles = jnp.cumsum(tiles)
    off = jnp.concatenate([jnp.zeros((1,), jnp.int32),
                           (cum_tiles[:-1] * BM).astype(jnp.int32)])
    pos = jnp.take(off, ti) + rank
    eid = jnp.clip(
        jnp.sum(jnp.arange(NT, dtype=jnp.int32)[:, None]
                >= cum_tiles[None, :].astype(jnp.int32), axis=1),
        0, E - 1).astype(jnp.int32)
    p0 = pos[:, 0].reshape(32, 4, 16)
    p1 = pos[:, 1].reshape(32, 4, 16)
    idx3 = jnp.concatenate([p0, p1], axis=1).astype(jnp.int32)

    xg = _sc_scatter_rows(x2, idx3, NTOT)
    shared_b = _shared_mlp_half(x2, params['sg_w'], params['su_w'],
                                params['sd_w'], 1)
    xg, shared_b = lax.optimization_barrier((xg, shared_b))
    yg = _grouped_mlp(eid, xg, params['eg_w'], params['eu_w'], params['ed_w'])
    pcat = jnp.concatenate([pos[:, 0], pos[:, 1]]).astype(jnp.int32)
    ypair = _sc_gather_rows(yg, pcat, TOPK * S)
    shared_a = _shared_mlp_half(x2, params['sg_w'], params['su_w'],
                                params['sd_w'], 0)
    out = _epilogue(hidden, shared_a, shared_b, sig, tv, ypair)
    return out.reshape(B, S, D)


# speedup vs baseline: 1.0695x; 1.0695x over previous
import functools

import jax
import jax.numpy as jnp
from jax import lax
from jax.experimental import pallas as pl
from jax.experimental.pallas import tpu as pltpu
from jax.experimental.pallas import tpu_sc as plsc

B, S, D = 1, 2048, 2048
H, Dh = 16, 128
E, TOPK = 8, 2
MOE_I, SHARED_I = 1408, 5632
EPS, THETA = 1e-6, 10000.0

BT = 256
BQ = 1024
BK = 1024
RQK = BQ // BK
BN = 256
BM = 128
NT = S * TOPK // BM + E
NTOT = NT * BM

_VMEM_BIG = pltpu.CompilerParams(vmem_limit_bytes=66846720)


def _rmsnorm_body(x_ref, w_ref, o_ref):
    x = x_ref[...]
    v = jnp.mean(x * x, axis=1, keepdims=True)
    o_ref[...] = x * lax.rsqrt(v + EPS) * w_ref[...]


def _rmsnorm(x, w):
    return pl.pallas_call(
        _rmsnorm_body,
        grid=(S // BT,),
        in_specs=[pl.BlockSpec((BT, D), lambda i: (i, 0)),
                  pl.BlockSpec((1, D), lambda i: (0, 0))],
        out_specs=pl.BlockSpec((BT, D), lambda i: (i, 0)),
        out_shape=jax.ShapeDtypeStruct((S, D), jnp.float32),
    )(x, w.reshape(1, D))


def _rope_tables_body(cos_ref, sin_ref):
    pos = lax.broadcasted_iota(jnp.int32, (S, 1), 0).astype(jnp.float32)
    inv = THETA ** (
        -2.0 * lax.broadcasted_iota(jnp.int32, (1, Dh // 2), 1).astype(jnp.float32) / Dh)
    freqs = pos * inv
    cos_h, sin_h = jnp.cos(freqs), jnp.sin(freqs)
    cos_ref[...] = jnp.concatenate([cos_h, cos_h], axis=1)
    sin_ref[...] = jnp.concatenate([sin_h, sin_h], axis=1)


def _rope_tables():
    return pl.pallas_call(
        _rope_tables_body,
        out_shape=[jax.ShapeDtypeStruct((S, Dh), jnp.float32),
                   jax.ShapeDtypeStruct((S, Dh), jnp.float32)],
    )()


BNQ = 256


def _apply_rope(acc, cos, sin):
    outs = []
    for h in range(BNQ // Dh):
        qh = acc[:, h * Dh:(h + 1) * Dh]
        x1, x2 = qh[:, :Dh // 2], qh[:, Dh // 2:]
        rot = jnp.concatenate([-x2, x1], axis=1)
        outs.append(qh * cos + rot * sin)
    return jnp.concatenate(outs, axis=1)


def _qkv_body(x_ref, ln_ref, qw_ref, kw_ref, vw_ref, cos_ref, sin_ref,
              q_ref, k_ref, v_ref):
    x = x_ref[...]
    var = jnp.mean(x * x, axis=1, keepdims=True)
    xn = x * lax.rsqrt(var + EPS) * ln_ref[...]
    cos, sin = cos_ref[...], sin_ref[...]
    q = jnp.dot(xn, qw_ref[...], preferred_element_type=jnp.float32)
    q_ref[...] = _apply_rope(q, cos, sin)
    k = jnp.dot(xn, kw_ref[...], preferred_element_type=jnp.float32)
    k_ref[...] = _apply_rope(k, cos, sin)
    v_ref[...] = jnp.dot(xn, vw_ref[...], preferred_element_type=jnp.float32)


def _qkv_project(x, ln_w, q_w, k_w, v_w, cos, sin):
    wspec = pl.BlockSpec((D, BNQ), lambda j: (0, j))
    ospec = pl.BlockSpec((S, BNQ), lambda j: (0, j))
    tspec = pl.BlockSpec((S, Dh), lambda j: (0, 0))
    return pl.pallas_call(
        _qkv_body,
        grid=(D // BNQ,),
        in_specs=[pl.BlockSpec((S, D), lambda j: (0, 0)),
                  pl.BlockSpec((1, D), lambda j: (0, 0)),
                  wspec, wspec, wspec, tspec, tspec],
        out_specs=[ospec, ospec, ospec],
        out_shape=[jax.ShapeDtypeStruct((S, D), jnp.float32)] * 3,
        compiler_params=_VMEM_BIG,
    )(x, ln_w.reshape(1, D), q_w, k_w, v_w, cos, sin)


def _flash_body(qk_ref, q_ref, k_ref, v_ref, o_ref, acc_ref, m_ref, l_ref):
    t = pl.program_id(1)
    qi, ki = qk_ref[0, t], qk_ref[1, t]

    @pl.when(ki == 0)
    def _():
        acc_ref[...] = jnp.zeros_like(acc_ref)
        m_ref[...] = jnp.full_like(m_ref, -jnp.inf)
        l_ref[...] = jnp.zeros_like(l_ref)

    q, k, v = q_ref[...], k_ref[...], v_ref[...]
    s = lax.dot_general(q, k, (((1,), (1,)), ((), ())),
                        preferred_element_type=jnp.float32)
    s = s * (1.0 / (Dh ** 0.5))

    def _update(s):
        m_prev, l_prev = m_ref[...], l_ref[...]
        m_cur = jnp.max(s, axis=1, keepdims=True)
        m_new = jnp.maximum(m_prev, m_cur)
        alpha = jnp.exp(m_prev - m_new)
        p = jnp.exp(s - m_new[:, 0:1])
        l_ref[...] = alpha * l_prev + jnp.sum(p, axis=1, keepdims=True)
        acc_ref[...] = alpha * acc_ref[...] + jnp.dot(
            p, v, preferred_element_type=jnp.float32)
        m_ref[...] = m_new

    @pl.when(ki < qi * RQK)
    def _():
        _update(s)

    @pl.when(ki >= qi * RQK)
    def _():
        row = qi * BQ + lax.broadcasted_iota(jnp.int32, (BQ, BK), 0)
        col = ki * BK + lax.broadcasted_iota(jnp.int32, (BQ, BK), 1)
        _update(jnp.where(col <= row, s, -1e30))

    @pl.when(ki == (qi + 1) * RQK - 1)
    def _():
        o_ref[...] = acc_ref[...] / l_ref[...]


def _flash_attention(q, k, v):
    nq = S // BQ
    tri = [(qi, ki) for qi in range(nq) for ki in range((qi + 1) * RQK)]
    qk = jnp.array([[a for a, _ in tri], [b for _, b in tri]], jnp.int32)
    spec = pltpu.PrefetchScalarGridSpec(
        num_scalar_prefetch=1,
        grid=(H, len(tri)),
        in_specs=[
            pl.BlockSpec((BQ, Dh), lambda h, t, qk: (qk[0, t], h)),
            pl.BlockSpec((BK, Dh), lambda h, t, qk: (qk[1, t], h)),
            pl.BlockSpec((BK, Dh), lambda h, t, qk: (qk[1, t], h)),
        ],
        out_specs=pl.BlockSpec((BQ, Dh), lambda h, t, qk: (qk[0, t], h)),
        scratch_shapes=[pltpu.VMEM((BQ, Dh), jnp.float32),
                        pltpu.VMEM((BQ, 128), jnp.float32),
                        pltpu.VMEM((BQ, 128), jnp.float32)],
    )
    return pl.pallas_call(
        _flash_body,
        grid_spec=spec,
        out_shape=jax.ShapeDtypeStruct((S, H * Dh), jnp.float32),
    )(qk, q, k, v)


def _mm_res_body(x_ref, w_ref, r_ref, o_ref):
    o_ref[...] = r_ref[...] + jnp.dot(x_ref[...], w_ref[...],
                                      preferred_element_type=jnp.float32)


def _o_proj_residual(ctx, o_w, residual):
    return pl.pallas_call(
        _mm_res_body,
        grid=(D // BN,),
        in_specs=[pl.BlockSpec((S, H * Dh), lambda j: (0, 0)),
                  pl.BlockSpec((H * Dh, BN), lambda j: (0, j)),
                  pl.BlockSpec((S, BN), lambda j: (0, j))],
        out_specs=pl.BlockSpec((S, BN), lambda j: (0, j)),
        out_shape=jax.ShapeDtypeStruct((S, D), jnp.float32),
        compiler_params=_VMEM_BIG,
    )(ctx, o_w, residual)


def _router_body(x_ref, ln_ref, rw_ref, gw_ref,
                 x2_ref, tv_ref, ti_ref, sig_ref):
    x = x_ref[...]
    v = jnp.mean(x * x, axis=1, keepdims=True)
    xn = x * lax.rsqrt(v + EPS) * ln_ref[...]
    x2_ref[...] = xn
    logits = jnp.dot(xn, rw_ref[...], preferred_element_type=jnp.float32)
    m = jnp.max(logits, axis=1, keepdims=True)
    e = jnp.exp(logits - m)
    p = e / jnp.sum(e, axis=1, keepdims=True)
    ii = lax.broadcasted_iota(jnp.int32, p.shape, 1)
    m1 = jnp.max(p, axis=1, keepdims=True)
    i1 = jnp.min(jnp.where(p == m1, ii, E), axis=1, keepdims=True)
    p2 = jnp.where(ii == i1, -1.0, p)
    m2 = jnp.max(p2, axis=1, keepdims=True)
    i2 = jnp.min(jnp.where(p2 == m2, ii, E), axis=1, keepdims=True)
    tv_ref[...] = jnp.concatenate([m1, m2], axis=1)
    ti_ref[...] = jnp.concatenate([i1, i2], axis=1)
    sg = jnp.dot(xn, gw_ref[...], preferred_element_type=jnp.float32)
    sig_ref[...] = jax.nn.sigmoid(sg)


def _router(hidden, ln2_w, router_w, sgate_w):
    return pl.pallas_call(
        _router_body,
        grid=(S // BT,),
        in_specs=[pl.BlockSpec((BT, D), lambda i: (i, 0)),
                  pl.BlockSpec((1, D), lambda i: (0, 0)),
                  pl.BlockSpec((D, E), lambda i: (0, 0)),
                  pl.BlockSpec((D, 1), lambda i: (0, 0))],
        out_specs=[pl.BlockSpec((BT, D), lambda i: (i, 0)),
                   pl.BlockSpec((BT, TOPK), lambda i: (i, 0)),
                   pl.BlockSpec((BT, TOPK), lambda i: (i, 0)),
                   pl.BlockSpec((BT, 1), lambda i: (i, 0))],
        out_shape=[jax.ShapeDtypeStruct((S, D), jnp.float32),
                   jax.ShapeDtypeStruct((S, TOPK), jnp.float32),
                   jax.ShapeDtypeStruct((S, TOPK), jnp.int32),
                   jax.ShapeDtypeStruct((S, 1), jnp.float32)],
    )(hidden, ln2_w.reshape(1, D), router_w, sgate_w)


def _sc_gather_rows(table, idx, total):
    info = plsc.get_sparse_core_info()
    nw = info.num_cores * info.num_subcores
    bpw = total // nw
    ch = 16
    dm = table.shape[1]
    mesh = plsc.VectorSubcoreMesh(core_axis_name="c", subcore_axis_name="s")

    @functools.partial(
        pl.kernel, mesh=mesh,
        out_type=jax.ShapeDtypeStruct((total, dm), jnp.float32),
        scratch_types=[pltpu.VMEM((bpw,), jnp.int32),
                       pltpu.VMEM((ch, dm), jnp.float32),
                       pltpu.VMEM((ch, dm), jnp.float32),
                       pltpu.SemaphoreType.DMA,
                       pltpu.SemaphoreType.DMA],
    )
    def gather(table_hbm, idx_hbm, out_hbm, idx_v, rows_a, rows_b, sem_a, sem_b):
        wid = lax.axis_index("s") * info.num_cores + lax.axis_index("c")
        base = wid * bpw
        pltpu.sync_copy(idx_hbm.at[pl.ds(base, bpw)], idx_v)
        bufs = (rows_a, rows_b)
        sems = (sem_a, sem_b)
        nch = bpw // ch
        cps = [None, None]
        for c in range(nch):
            cps[c % 2] = pltpu.async_copy(
                table_hbm.at[idx_v.at[pl.ds(c * ch, ch)]], bufs[c % 2],
                sems[c % 2])
            if c > 0:
                cps[(c - 1) % 2].wait()
                pltpu.sync_copy(bufs[(c - 1) % 2],
                                out_hbm.at[pl.ds(base + (c - 1) * ch, ch)])
        cps[(nch - 1) % 2].wait()
        pltpu.sync_copy(bufs[(nch - 1) % 2],
                        out_hbm.at[pl.ds(base + (nch - 1) * ch, ch)])

    return gather(table, idx)


def _sc_scatter_rows(rows, idx3, total):
    info = plsc.get_sparse_core_info()
    nw = info.num_cores * info.num_subcores
    nwk, nch, ch = idx3.shape
    rpw = rows.shape[0] // nw
    dm = rows.shape[1]
    mesh = plsc.VectorSubcoreMesh(core_axis_name="c", subcore_axis_name="s")

    @functools.partial(
        pl.kernel, mesh=mesh,
        out_type=jax.ShapeDtypeStruct((total, dm), jnp.float32),
        scratch_types=[pltpu.VMEM((nch, ch), jnp.int32),
                       pltpu.VMEM((ch, dm), jnp.float32),
                       pltpu.VMEM((ch, dm), jnp.float32),
                       pltpu.SemaphoreType.DMA,
                       pltpu.SemaphoreType.DMA],
    )
    def scatter(rows_hbm, idx_hbm, out_hbm, idx_v, buf_a, buf_b, sem_a, sem_b):
        wid = lax.axis_index("s") * info.num_cores + lax.axis_index("c")
        base = wid * rpw
        pltpu.sync_copy(idx_hbm.at[wid], idx_v)
        bufs = (buf_a, buf_b)
        sems = (sem_a, sem_b)
        half = nch // 2
        cps = [None, None]
        for c in range(nch):
            if cps[c % 2] is not None:
                cps[c % 2].wait()
            src = base + (c % half) * ch
            pltpu.sync_copy(rows_hbm.at[pl.ds(src, ch)], bufs[c % 2])
            cps[c % 2] = pltpu.async_copy(
                bufs[c % 2], out_hbm.at[idx_v.at[c]], sems[c % 2])
        cps[(nch - 1) % 2].wait()
        if nch > 1:
            cps[(nch - 2) % 2].wait()

    return scatter(rows, idx3)


def _moe_gu_body(eid_ref, xg_ref, eg_ref, eu_ref, h_ref):
    x = xg_ref[...]
    g = jnp.dot(x, eg_ref[0], preferred_element_type=jnp.float32)
    u = jnp.dot(x, eu_ref[0], preferred_element_type=jnp.float32)
    h_ref[...] = g * jax.nn.sigmoid(g) * u


def _moe_down_body(eid_ref, h_ref, ed_ref, yg_ref):
    yg_ref[...] = jnp.dot(h_ref[...], ed_ref[0],
                          preferred_element_type=jnp.float32)


def _grouped_mlp(eid, xg, eg_w, eu_w, ed_w):
    gu_spec = pltpu.PrefetchScalarGridSpec(
        num_scalar_prefetch=1,
        grid=(NT,),
        in_specs=[
            pl.BlockSpec((BM, D), lambda t, eid: (t, 0)),
            pl.BlockSpec((1, D, MOE_I), lambda t, eid: (eid[t], 0, 0)),
            pl.BlockSpec((1, D, MOE_I), lambda t, eid: (eid[t], 0, 0)),
        ],
        out_specs=pl.BlockSpec((BM, MOE_I), lambda t, eid: (t, 0)),
    )
    h = pl.pallas_call(
        _moe_gu_body,
        grid_spec=gu_spec,
        out_shape=jax.ShapeDtypeStruct((NTOT, MOE_I), jnp.float32),
        compiler_params=_VMEM_BIG,
    )(eid, xg, eg_w, eu_w)
    down_spec = pltpu.PrefetchScalarGridSpec(
        num_scalar_prefetch=1,
        grid=(NT,),
        in_specs=[
            pl.BlockSpec((BM, MOE_I), lambda t, eid: (t, 0)),
            pl.BlockSpec((1, MOE_I, D), lambda t, eid: (eid[t], 0, 0)),
        ],
        out_specs=pl.BlockSpec((BM, D), lambda t, eid: (t, 0)),
    )
    return pl.pallas_call(
        _moe_down_body,
        grid_spec=down_spec,
        out_shape=jax.ShapeDtypeStruct((NTOT, D), jnp.float32),
        compiler_params=_VMEM_BIG,
    )(eid, h, ed_w)


def _shared_body(x_ref, g_ref, u_ref, d_ref, o_ref):
    j = pl.program_id(0)
    x = x_ref[...]
    g = jnp.dot(x, g_ref[...], preferred_element_type=jnp.float32)
    u = jnp.dot(x, u_ref[...], preferred_element_type=jnp.float32)
    h = g * jax.nn.sigmoid(g) * u
    y = jnp.dot(h, d_ref[...], preferred_element_type=jnp.float32)

    @pl.when(j == 0)
    def _():
        o_ref[...] = y

    @pl.when(j > 0)
    def _():
        o_ref[...] = o_ref[...] + y


BMS = 1024


def _shared_mlp_half(x2, sg_w, su_w, sd_w, half):
    return pl.pallas_call(
        _shared_body,
        grid=(SHARED_I // BN,),
        in_specs=[pl.BlockSpec((BMS, D), lambda j, h=half: (h, 0)),
                  pl.BlockSpec((D, BN), lambda j: (0, j)),
                  pl.BlockSpec((D, BN), lambda j: (0, j)),
                  pl.BlockSpec((BN, D), lambda j: (j, 0))],
        out_specs=pl.BlockSpec((BMS, D), lambda j: (0, 0)),
        out_shape=jax.ShapeDtypeStruct((BMS, D), jnp.float32),
        compiler_params=_VMEM_BIG,
    )(x2, sg_w, su_w, sd_w)


def _epilogue_body(r_ref, sha_ref, shb_ref, sig_ref, tv_ref, y1_ref, y2_ref,
                   o_ref):
    i = pl.program_id(0)
    w = tv_ref[...]
    sh = jnp.where(i < (BMS // BT), sha_ref[...], shb_ref[...])
    o_ref[...] = (r_ref[...] + sh * sig_ref[...]
                  + w[:, 0:1] * y1_ref[...] + w[:, 1:2] * y2_ref[...])


def _epilogue(residual, shared_a, shared_b, sig, tv, ypair):
    nb = S // BT
    hb = BMS // BT
    return pl.pallas_call(
        _epilogue_body,
        grid=(nb,),
        in_specs=[pl.BlockSpec((BT, D), lambda i: (i, 0)),
                  pl.BlockSpec((BT, D), lambda i, hb=hb: (jnp.minimum(i, hb - 1), 0)),
                  pl.BlockSpec((BT, D), lambda i, hb=hb: (jnp.maximum(i - hb, 0), 0)),
                  pl.BlockSpec((BT, 1), lambda i: (i, 0)),
                  pl.BlockSpec((BT, TOPK), lambda i: (i, 0)),
                  pl.BlockSpec((BT, D), lambda i: (i, 0)),
                  pl.BlockSpec((BT, D), lambda i, nb=nb: (i + nb, 0))],
        out_specs=pl.BlockSpec((BT, D), lambda i: (i, 0)),
        out_shape=jax.ShapeDtypeStruct((S, D), jnp.float32),
    )(residual, shared_a, shared_b, sig, tv, ypair, ypair)


def kernel(hidden_states, attention_mask, position_ids, causal_mask, params):
    x = hidden_states.reshape(S, D).astype(jnp.float32)

    cos, sin = _rope_tables()
    q, k, v = _qkv_project(x, params['ln1_w'], params['q_w'], params['k_w'],
                           params['v_w'], cos, sin)
    ctx = _flash_attention(q, k, v)
    hidden = _o_proj_residual(ctx, params['o_w'], x)

    x2, tv, ti, sig = _router(hidden, params['ln2_w'],
                              params['router_w'], params['sgate_w'])

    onehot = (ti[:, :, None] == jnp.arange(E)[None, None, :]).astype(jnp.int32)
    per_tok = onehot.sum(1)
    counts = per_tok.sum(0)
    excl = jnp.cumsum(per_tok, axis=0) - per_tok
    rank = jnp.take_along_axis(excl, ti, axis=1)
    tiles = (counts + BM - 1) // BM
    cum_tiles = jnp.cumsum(tiles)
    off = jnp.concatenate([jnp.zeros((1,), jnp.int32),
                           (cum_tiles[:-1] * BM).astype(jnp.int32)])
    pos = jnp.take(off, ti) + rank
    eid = jnp.clip(
        jnp.sum(jnp.arange(NT, dtype=jnp.int32)[:, None]
                >= cum_tiles[None, :].astype(jnp.int32), axis=1),
        0, E - 1).astype(jnp.int32)
    p0 = pos[:, 0].reshape(32, 4, 16)
    p1 = pos[:, 1].reshape(32, 4, 16)
    idx3 = jnp.concatenate([p0, p1], axis=1).astype(jnp.int32)

    xg = _sc_scatter_rows(x2, idx3, NTOT)
    shared_b = _shared_mlp_half(x2, params['sg_w'], params['su_w'],
                                params['sd_w'], 1)
    xg, shared_b = lax.optimization_barrier((xg, shared_b))
    yg = _grouped_mlp(eid, xg, params['eg_w'], params['eu_w'], params['ed_w'])
    pcat = jnp.concatenate([pos[:, 0], pos[:, 1]]).astype(jnp.int32)
    ypair = _sc_gather_rows(yg, pcat, TOPK * S)
    shared_a = _shared_mlp_half(x2, params['sg_w'], params['su_w'],
                                params['sd_w'], 0)
    out = _epilogue(hidden, shared_a, shared_b, sig, tv, ypair)
    return out.reshape(B, S, D)


# speedup vs baseline: 1.0886x; 1.0179x over previous
import functools

import jax
import jax.numpy as jnp
from jax import lax
from jax.experimental import pallas as pl
from jax.experimental.pallas import tpu as pltpu
from jax.experimental.pallas import tpu_sc as plsc

B, S, D = 1, 2048, 2048
H, Dh = 16, 128
E, TOPK = 8, 2
MOE_I, SHARED_I = 1408, 5632
EPS, THETA = 1e-6, 10000.0

BT = 256
BQ = 1024
BK = 1024
RQK = BQ // BK
BN = 256
BM = 128
NT = S * TOPK // BM + E
NTOT = NT * BM

_VMEM_BIG = pltpu.CompilerParams(vmem_limit_bytes=66846720)


def _rmsnorm_body(x_ref, w_ref, o_ref):
    x = x_ref[...]
    v = jnp.mean(x * x, axis=1, keepdims=True)
    o_ref[...] = x * lax.rsqrt(v + EPS) * w_ref[...]


def _rmsnorm(x, w):
    return pl.pallas_call(
        _rmsnorm_body,
        grid=(S // BT,),
        in_specs=[pl.BlockSpec((BT, D), lambda i: (i, 0)),
                  pl.BlockSpec((1, D), lambda i: (0, 0))],
        out_specs=pl.BlockSpec((BT, D), lambda i: (i, 0)),
        out_shape=jax.ShapeDtypeStruct((S, D), jnp.float32),
    )(x, w.reshape(1, D))


def _rope_tables_body(cos_ref, sin_ref):
    pos = lax.broadcasted_iota(jnp.int32, (S, 1), 0).astype(jnp.float32)
    inv = THETA ** (
        -2.0 * lax.broadcasted_iota(jnp.int32, (1, Dh // 2), 1).astype(jnp.float32) / Dh)
    freqs = pos * inv
    cos_h, sin_h = jnp.cos(freqs), jnp.sin(freqs)
    cos_ref[...] = jnp.concatenate([cos_h, cos_h], axis=1)
    sin_ref[...] = jnp.concatenate([sin_h, sin_h], axis=1)


def _rope_tables():
    return pl.pallas_call(
        _rope_tables_body,
        out_shape=[jax.ShapeDtypeStruct((S, Dh), jnp.float32),
                   jax.ShapeDtypeStruct((S, Dh), jnp.float32)],
    )()


BNQ = 256


def _apply_rope(acc, cos, sin):
    outs = []
    for h in range(BNQ // Dh):
        qh = acc[:, h * Dh:(h + 1) * Dh]
        x1, x2 = qh[:, :Dh // 2], qh[:, Dh // 2:]
        rot = jnp.concatenate([-x2, x1], axis=1)
        outs.append(qh * cos + rot * sin)
    return jnp.concatenate(outs, axis=1)


def _qkv_body(x_ref, ln_ref, qw_ref, kw_ref, vw_ref, cos_ref, sin_ref,
              q_ref, k_ref, v_ref):
    x = x_ref[...]
    var = jnp.mean(x * x, axis=1, keepdims=True)
    xn = x * lax.rsqrt(var + EPS) * ln_ref[...]
    cos, sin = cos_ref[...], sin_ref[...]
    q = jnp.dot(xn, qw_ref[...], preferred_element_type=jnp.float32)
    q_ref[...] = _apply_rope(q, cos, sin)
    k = jnp.dot(xn, kw_ref[...], preferred_element_type=jnp.float32)
    k_ref[...] = _apply_rope(k, cos, sin)
    v_ref[...] = jnp.dot(xn, vw_ref[...], preferred_element_type=jnp.float32)


def _qkv_project(x, ln_w, q_w, k_w, v_w, cos, sin):
    wspec = pl.BlockSpec((D, BNQ), lambda j: (0, j))
    ospec = pl.BlockSpec((S, BNQ), lambda j: (0, j))
    tspec = pl.BlockSpec((S, Dh), lambda j: (0, 0))
    return pl.pallas_call(
        _qkv_body,
        grid=(D // BNQ,),
        in_specs=[pl.BlockSpec((S, D), lambda j: (0, 0)),
                  pl.BlockSpec((1, D), lambda j: (0, 0)),
                  wspec, wspec, wspec, tspec, tspec],
        out_specs=[ospec, ospec, ospec],
        out_shape=[jax.ShapeDtypeStruct((S, D), jnp.float32)] * 3,
        compiler_params=_VMEM_BIG,
    )(x, ln_w.reshape(1, D), q_w, k_w, v_w, cos, sin)


def _flash_body(qk_ref, q_ref, k_ref, v_ref, o_ref, acc_ref, m_ref, l_ref):
    t = pl.program_id(1)
    qi, ki = qk_ref[0, t], qk_ref[1, t]

    @pl.when(ki == 0)
    def _():
        acc_ref[...] = jnp.zeros_like(acc_ref)
        m_ref[...] = jnp.full_like(m_ref, -jnp.inf)
        l_ref[...] = jnp.zeros_like(l_ref)

    q, k, v = q_ref[...], k_ref[...], v_ref[...]
    s = lax.dot_general(q, k, (((1,), (1,)), ((), ())),
                        preferred_element_type=jnp.float32)
    s = s * (1.0 / (Dh ** 0.5))

    def _update(s):
        m_prev, l_prev = m_ref[...], l_ref[...]
        m_cur = jnp.max(s, axis=1, keepdims=True)
        m_new = jnp.maximum(m_prev, m_cur)
        alpha = jnp.exp(m_prev - m_new)
        p = jnp.exp(s - m_new[:, 0:1])
        l_ref[...] = alpha * l_prev + jnp.sum(p, axis=1, keepdims=True)
        acc_ref[...] = alpha * acc_ref[...] + jnp.dot(
            p, v, preferred_element_type=jnp.float32)
        m_ref[...] = m_new

    @pl.when(ki < qi * RQK)
    def _():
        _update(s)

    @pl.when(ki >= qi * RQK)
    def _():
        row = qi * BQ + lax.broadcasted_iota(jnp.int32, (BQ, BK), 0)
        col = ki * BK + lax.broadcasted_iota(jnp.int32, (BQ, BK), 1)
        _update(jnp.where(col <= row, s, -1e30))

    @pl.when(ki == (qi + 1) * RQK - 1)
    def _():
        o_ref[...] = acc_ref[...] / l_ref[...]


def _flash_attention(q, k, v):
    nq = S // BQ
    tri = [(qi, ki) for qi in range(nq) for ki in range((qi + 1) * RQK)]
    qk = jnp.array([[a for a, _ in tri], [b for _, b in tri]], jnp.int32)
    spec = pltpu.PrefetchScalarGridSpec(
        num_scalar_prefetch=1,
        grid=(H, len(tri)),
        in_specs=[
            pl.BlockSpec((BQ, Dh), lambda h, t, qk: (qk[0, t], h)),
            pl.BlockSpec((BK, Dh), lambda h, t, qk: (qk[1, t], h)),
            pl.BlockSpec((BK, Dh), lambda h, t, qk: (qk[1, t], h)),
        ],
        out_specs=pl.BlockSpec((BQ, Dh), lambda h, t, qk: (qk[0, t], h)),
        scratch_shapes=[pltpu.VMEM((BQ, Dh), jnp.float32),
                        pltpu.VMEM((BQ, 128), jnp.float32),
                        pltpu.VMEM((BQ, 128), jnp.float32)],
    )
    return pl.pallas_call(
        _flash_body,
        grid_spec=spec,
        out_shape=jax.ShapeDtypeStruct((S, H * Dh), jnp.float32),
    )(qk, q, k, v)


def _mm_res_body(x_ref, w_ref, r_ref, o_ref):
    o_ref[...] = r_ref[...] + jnp.dot(x_ref[...], w_ref[...],
                                      preferred_element_type=jnp.float32)


def _o_proj_residual(ctx, o_w, residual):
    return pl.pallas_call(
        _mm_res_body,
        grid=(D // BN,),
        in_specs=[pl.BlockSpec((S, H * Dh), lambda j: (0, 0)),
                  pl.BlockSpec((H * Dh, BN), lambda j: (0, j)),
                  pl.BlockSpec((S, BN), lambda j: (0, j))],
        out_specs=pl.BlockSpec((S, BN), lambda j: (0, j)),
        out_shape=jax.ShapeDtypeStruct((S, D), jnp.float32),
        compiler_params=_VMEM_BIG,
    )(ctx, o_w, residual)


def _router_body(x_ref, ln_ref, rw_ref, gw_ref,
                 x2_ref, tv_ref, ti_ref, sig_ref):
    x = x_ref[...]
    v = jnp.mean(x * x, axis=1, keepdims=True)
    xn = x * lax.rsqrt(v + EPS) * ln_ref[...]
    x2_ref[...] = xn
    logits = jnp.dot(xn, rw_ref[...], preferred_element_type=jnp.float32)
    m = jnp.max(logits, axis=1, keepdims=True)
    e = jnp.exp(logits - m)
    p = e / jnp.sum(e, axis=1, keepdims=True)
    ii = lax.broadcasted_iota(jnp.int32, p.shape, 1)
    m1 = jnp.max(p, axis=1, keepdims=True)
    i1 = jnp.min(jnp.where(p == m1, ii, E), axis=1, keepdims=True)
    p2 = jnp.where(ii == i1, -1.0, p)
    m2 = jnp.max(p2, axis=1, keepdims=True)
    i2 = jnp.min(jnp.where(p2 == m2, ii, E), axis=1, keepdims=True)
    tv_ref[...] = jnp.concatenate([m1, m2], axis=1)
    ti_ref[...] = jnp.concatenate([i1, i2], axis=1)
    sg = jnp.dot(xn, gw_ref[...], preferred_element_type=jnp.float32)
    sig_ref[...] = jax.nn.sigmoid(sg)


def _router(hidden, ln2_w, router_w, sgate_w):
    return pl.pallas_call(
        _router_body,
        grid=(S // BT,),
        in_specs=[pl.BlockSpec((BT, D), lambda i: (i, 0)),
                  pl.BlockSpec((1, D), lambda i: (0, 0)),
                  pl.BlockSpec((D, E), lambda i: (0, 0)),
                  pl.BlockSpec((D, 1), lambda i: (0, 0))],
        out_specs=[pl.BlockSpec((BT, D), lambda i: (i, 0)),
                   pl.BlockSpec((BT, TOPK), lambda i: (i, 0)),
                   pl.BlockSpec((BT, TOPK), lambda i: (i, 0)),
                   pl.BlockSpec((BT, 1), lambda i: (i, 0))],
        out_shape=[jax.ShapeDtypeStruct((S, D), jnp.float32),
                   jax.ShapeDtypeStruct((S, TOPK), jnp.float32),
                   jax.ShapeDtypeStruct((S, TOPK), jnp.int32),
                   jax.ShapeDtypeStruct((S, 1), jnp.float32)],
    )(hidden, ln2_w.reshape(1, D), router_w, sgate_w)


def _sc_gather_rows(table, idx, total):
    info = plsc.get_sparse_core_info()
    nw = info.num_cores * info.num_subcores
    bpw = total // nw
    ch = 16
    dm = table.shape[1]
    mesh = plsc.VectorSubcoreMesh(core_axis_name="c", subcore_axis_name="s")

    @functools.partial(
        pl.kernel, mesh=mesh,
        out_type=jax.ShapeDtypeStruct((total, dm), jnp.float32),
        scratch_types=[pltpu.VMEM((bpw,), jnp.int32),
                       pltpu.VMEM((ch, dm), jnp.float32),
                       pltpu.VMEM((ch, dm), jnp.float32),
                       pltpu.SemaphoreType.DMA,
                       pltpu.SemaphoreType.DMA],
    )
    def gather(table_hbm, idx_hbm, out_hbm, idx_v, rows_a, rows_b, sem_a, sem_b):
        wid = lax.axis_index("s") * info.num_cores + lax.axis_index("c")
        base = wid * bpw
        pltpu.sync_copy(idx_hbm.at[pl.ds(base, bpw)], idx_v)
        bufs = (rows_a, rows_b)
        sems = (sem_a, sem_b)
        nch = bpw // ch
        cps = [None, None]
        for c in range(nch):
            cps[c % 2] = pltpu.async_copy(
                table_hbm.at[idx_v.at[pl.ds(c * ch, ch)]], bufs[c % 2],
                sems[c % 2])
            if c > 0:
                cps[(c - 1) % 2].wait()
                pltpu.sync_copy(bufs[(c - 1) % 2],
                                out_hbm.at[pl.ds(base + (c - 1) * ch, ch)])
        cps[(nch - 1) % 2].wait()
        pltpu.sync_copy(bufs[(nch - 1) % 2],
                        out_hbm.at[pl.ds(base + (nch - 1) * ch, ch)])

    return gather(table, idx)


def _sc_scatter_rows(rows, idx3, total):
    info = plsc.get_sparse_core_info()
    nw = info.num_cores * info.num_subcores
    nwk, nch, ch = idx3.shape
    rpw = rows.shape[0] // nw
    dm = rows.shape[1]
    mesh = plsc.VectorSubcoreMesh(core_axis_name="c", subcore_axis_name="s")

    @functools.partial(
        pl.kernel, mesh=mesh,
        out_type=jax.ShapeDtypeStruct((total, dm), jnp.float32),
        scratch_types=[pltpu.VMEM((nch, ch), jnp.int32),
                       pltpu.VMEM((ch, dm), jnp.float32),
                       pltpu.VMEM((ch, dm), jnp.float32),
                       pltpu.SemaphoreType.DMA,
                       pltpu.SemaphoreType.DMA],
    )
    def scatter(rows_hbm, idx_hbm, out_hbm, idx_v, buf_a, buf_b, sem_a, sem_b):
        wid = lax.axis_index("s") * info.num_cores + lax.axis_index("c")
        base = wid * rpw
        pltpu.sync_copy(idx_hbm.at[wid], idx_v)
        bufs = (buf_a, buf_b)
        sems = (sem_a, sem_b)
        half = nch // 2
        cps = [None, None]
        for c in range(nch):
            if cps[c % 2] is not None:
                cps[c % 2].wait()
            src = base + (c % half) * ch
            pltpu.sync_copy(rows_hbm.at[pl.ds(src, ch)], bufs[c % 2])
            cps[c % 2] = pltpu.async_copy(
                bufs[c % 2], out_hbm.at[idx_v.at[c]], sems[c % 2])
        cps[(nch - 1) % 2].wait()
        if nch > 1:
            cps[(nch - 2) % 2].wait()

    return scatter(rows, idx3)


def _moe_gu_body(eid_ref, xg_ref, eg_ref, eu_ref, h_ref):
    x = xg_ref[...]
    g = jnp.dot(x, eg_ref[0], preferred_element_type=jnp.float32)
    u = jnp.dot(x, eu_ref[0], preferred_element_type=jnp.float32)
    h_ref[...] = g * jax.nn.sigmoid(g) * u


def _moe_down_body(eid_ref, h_ref, ed_ref, yg_ref):
    yg_ref[...] = jnp.dot(h_ref[...], ed_ref[0],
                          preferred_element_type=jnp.float32)


def _grouped_mlp(eid, xg, eg_w, eu_w, ed_w):
    gu_spec = pltpu.PrefetchScalarGridSpec(
        num_scalar_prefetch=1,
        grid=(NT,),
        in_specs=[
            pl.BlockSpec((BM, D), lambda t, eid: (t, 0)),
            pl.BlockSpec((1, D, MOE_I), lambda t, eid: (eid[t], 0, 0)),
            pl.BlockSpec((1, D, MOE_I), lambda t, eid: (eid[t], 0, 0)),
        ],
        out_specs=pl.BlockSpec((BM, MOE_I), lambda t, eid: (t, 0)),
    )
    h = pl.pallas_call(
        _moe_gu_body,
        grid_spec=gu_spec,
        out_shape=jax.ShapeDtypeStruct((NTOT, MOE_I), jnp.float32),
        compiler_params=_VMEM_BIG,
    )(eid, xg, eg_w, eu_w)
    down_spec = pltpu.PrefetchScalarGridSpec(
        num_scalar_prefetch=1,
        grid=(NT,),
        in_specs=[
            pl.BlockSpec((BM, MOE_I), lambda t, eid: (t, 0)),
            pl.BlockSpec((1, MOE_I, D), lambda t, eid: (eid[t], 0, 0)),
        ],
        out_specs=pl.BlockSpec((BM, D), lambda t, eid: (t, 0)),
    )
    return pl.pallas_call(
        _moe_down_body,
        grid_spec=down_spec,
        out_shape=jax.ShapeDtypeStruct((NTOT, D), jnp.float32),
        compiler_params=_VMEM_BIG,
    )(eid, h, ed_w)


def _shared_body(x_ref, g_ref, u_ref, d_ref, o_ref):
    j = pl.program_id(0)
    x = x_ref[...]
    g = jnp.dot(x, g_ref[...], preferred_element_type=jnp.float32)
    u = jnp.dot(x, u_ref[...], preferred_element_type=jnp.float32)
    h = g * jax.nn.sigmoid(g) * u
    y = jnp.dot(h, d_ref[...], preferred_element_type=jnp.float32)

    @pl.when(j == 0)
    def _():
        o_ref[...] = y

    @pl.when(j > 0)
    def _():
        o_ref[...] = o_ref[...] + y


BMS = 1024
BNS = 512


def _shared_mlp_half(x2, sg_w, su_w, sd_w, half):
    return pl.pallas_call(
        _shared_body,
        grid=(SHARED_I // BNS,),
        in_specs=[pl.BlockSpec((BMS, D), lambda j, h=half: (h, 0)),
                  pl.BlockSpec((D, BNS), lambda j: (0, j)),
                  pl.BlockSpec((D, BNS), lambda j: (0, j)),
                  pl.BlockSpec((BNS, D), lambda j: (j, 0))],
        out_specs=pl.BlockSpec((BMS, D), lambda j: (0, 0)),
        out_shape=jax.ShapeDtypeStruct((BMS, D), jnp.float32),
        compiler_params=_VMEM_BIG,
    )(x2, sg_w, su_w, sd_w)


def _epilogue_body(r_ref, sha_ref, shb_ref, sig_ref, tv_ref, y1_ref, y2_ref,
                   o_ref):
    i = pl.program_id(0)
    w = tv_ref[...]
    sh = jnp.where(i < (BMS // BT), sha_ref[...], shb_ref[...])
    o_ref[...] = (r_ref[...] + sh * sig_ref[...]
                  + w[:, 0:1] * y1_ref[...] + w[:, 1:2] * y2_ref[...])


def _epilogue(residual, shared_a, shared_b, sig, tv, ypair):
    nb = S // BT
    hb = BMS // BT
    return pl.pallas_call(
        _epilogue_body,
        grid=(nb,),
        in_specs=[pl.BlockSpec((BT, D), lambda i: (i, 0)),
                  pl.BlockSpec((BT, D), lambda i, hb=hb: (jnp.minimum(i, hb - 1), 0)),
                  pl.BlockSpec((BT, D), lambda i, hb=hb: (jnp.maximum(i - hb, 0), 0)),
                  pl.BlockSpec((BT, 1), lambda i: (i, 0)),
                  pl.BlockSpec((BT, TOPK), lambda i: (i, 0)),
                  pl.BlockSpec((BT, D), lambda i: (i, 0)),
                  pl.BlockSpec((BT, D), lambda i, nb=nb: (i + nb, 0))],
        out_specs=pl.BlockSpec((BT, D), lambda i: (i, 0)),
        out_shape=jax.ShapeDtypeStruct((S, D), jnp.float32),
    )(residual, shared_a, shared_b, sig, tv, ypair, ypair)


def kernel(hidden_states, attention_mask, position_ids, causal_mask, params):
    x = hidden_states.reshape(S, D).astype(jnp.float32)

    cos, sin = _rope_tables()
    q, k, v = _qkv_project(x, params['ln1_w'], params['q_w'], params['k_w'],
                           params['v_w'], cos, sin)
    ctx = _flash_attention(q, k, v)
    hidden = _o_proj_residual(ctx, params['o_w'], x)

    x2, tv, ti, sig = _router(hidden, params['ln2_w'],
                              params['router_w'], params['sgate_w'])

    onehot = (ti[:, :, None] == jnp.arange(E)[None, None, :]).astype(jnp.int32)
    per_tok = onehot.sum(1)
    counts = per_tok.sum(0)
    excl = jnp.cumsum(per_tok, axis=0) - per_tok
    rank = jnp.take_along_axis(excl, ti, axis=1)
    tiles = (counts + BM - 1) // BM
    cum_tiles = jnp.cumsum(tiles)
    off = jnp.concatenate([jnp.zeros((1,), jnp.int32),
                           (cum_tiles[:-1] * BM).astype(jnp.int32)])
    pos = jnp.take(off, ti) + rank
    eid = jnp.clip(
        jnp.sum(jnp.arange(NT, dtype=jnp.int32)[:, None]
                >= cum_tiles[None, :].astype(jnp.int32), axis=1),
        0, E - 1).astype(jnp.int32)
    p0 = pos[:, 0].reshape(32, 4, 16)
    p1 = pos[:, 1].reshape(32, 4, 16)
    idx3 = jnp.concatenate([p0, p1], axis=1).astype(jnp.int32)

    xg = _sc_scatter_rows(x2, idx3, NTOT)
    shared_b = _shared_mlp_half(x2, params['sg_w'], params['su_w'],
                                params['sd_w'], 1)
    xg, shared_b = lax.optimization_barrier((xg, shared_b))
    yg = _grouped_mlp(eid, xg, params['eg_w'], params['eu_w'], params['ed_w'])
    pcat = jnp.concatenate([pos[:, 0], pos[:, 1]]).astype(jnp.int32)
    ypair = _sc_gather_rows(yg, pcat, TOPK * S)
    shared_a = _shared_mlp_half(x2, params['sg_w'], params['su_w'],
                                params['sd_w'], 0)
    out = _epilogue(hidden, shared_a, shared_b, sig, tv, ypair)
    return out.reshape(B, S, D)
